# Initial kernel scaffold; baseline (speedup 1.0000x reference)
import jax
import jax.numpy as jnp
from jax.experimental import pallas as pl


def kernel(x, g1, b1, Wqkv, bqkv, Wproj, bproj, g2, b2, Wg, bg, Wfc, bfc, Wfp, bfp):
    raise NotImplementedError("write your pallas kernel here")


# speedup vs baseline: 2.3559x; 2.3559x over previous
import functools

import jax
import jax.numpy as jnp
from jax import lax
from jax.experimental import pallas as pl
from jax.experimental.pallas import tpu as pltpu, tpu_sc as plsc

T = 2048
N = 768
H = 12
DH = 64
E = 8
FF = 4 * N
RB = 256
QB = 1024
MB = 128
XG_ROWS = 5120
NBLK = XG_ROWS // MB


def _ln_qkv_body(x_ref, g_ref, b_ref, w_ref, bias_ref, o_ref):
    xb = x_ref[:]
    m = jnp.mean(xb, axis=1, keepdims=True)
    xc = xb - m
    v = jnp.mean(xc * xc, axis=1, keepdims=True)
    ln = xc * lax.rsqrt(v + 1e-5) * g_ref[:] + b_ref[:]
    o_ref[:] = (
        jnp.dot(ln, w_ref[:], preferred_element_type=jnp.float32) + bias_ref[:]
    )


def _ln_qkv(x2, g1, b1, Wqkv, bqkv, interpret=False):
    return pl.pallas_call(
        _ln_qkv_body,
        grid=(T // RB,),
        in_specs=[
            pl.BlockSpec((RB, N), lambda i: (i, 0)),
            pl.BlockSpec((1, N), lambda i: (0, 0)),
            pl.BlockSpec((1, N), lambda i: (0, 0)),
            pl.BlockSpec((N, 3 * N), lambda i: (0, 0)),
            pl.BlockSpec((1, 3 * N), lambda i: (0, 0)),
        ],
        out_specs=pl.BlockSpec((RB, 3 * N), lambda i: (i, 0)),
        out_shape=jax.ShapeDtypeStruct((T, 3 * N), jnp.float32),
        interpret=interpret,
    )(x2, g1.reshape(1, N), b1.reshape(1, N), Wqkv, bqkv.reshape(1, 3 * N))


def _attn_body(q_ref, k_ref, v_ref, o_ref, m_s, l_s):
    i = pl.program_id(1)
    j = pl.program_id(2)

    @pl.when(j == 0)
    def _():
        m_s[:] = jnp.full((QB, 1), -jnp.inf, jnp.float32)
        l_s[:] = jnp.zeros((QB, 1), jnp.float32)
        o_ref[0] = jnp.zeros((QB, DH), jnp.float32)

    s = lax.dot_general(q_ref[0], k_ref[0], (((1,), (1,)), ((), ())),
                        preferred_element_type=jnp.float32)
    s = s * 0.125
    row = i * QB + lax.broadcasted_iota(jnp.int32, (QB, QB), 0)
    col = j * QB + lax.broadcasted_iota(jnp.int32, (QB, QB), 1)
    s = jnp.where(col <= row, s, jnp.finfo(jnp.float32).min)
    bm = jnp.max(s, axis=1, keepdims=True)
    m_old = m_s[:]
    m_new = jnp.maximum(m_old, bm)
    delta = jnp.where(m_old == m_new, jnp.float32(0.0), m_old - m_new)
    e = jnp.exp(s - m_new)
    bs = jnp.sum(e, axis=1, keepdims=True)
    l_old = l_s[:]
    ed = jnp.exp(delta)
    scale = ed * l_old
    l_new = scale + bs
    acc = scale * o_ref[0]
    mm = jnp.dot(e, v_ref[0], preferred_element_type=jnp.float32) + acc
    o_ref[0] = mm * (jnp.float32(1.0) / l_new)
    m_s[:] = m_new
    l_s[:] = l_new


def _attn(qkvT, interpret=False):
    return pl.pallas_call(
        _attn_body,
        grid=(H, T // QB, T // QB),
        in_specs=[
            pl.BlockSpec((1, QB, DH), lambda h, i, j: (h, i, 0)),
            pl.BlockSpec((1, QB, DH), lambda h, i, j: (H + h, j, 0)),
            pl.BlockSpec((1, QB, DH), lambda h, i, j: (2 * H + h, j, 0)),
        ],
        out_specs=pl.BlockSpec((1, QB, DH), lambda h, i, j: (h, i, 0)),
        out_shape=jax.ShapeDtypeStruct((H, T, DH), jnp.float32),
        scratch_shapes=[
            pltpu.VMEM((QB, 1), jnp.float32),
            pltpu.VMEM((QB, 1), jnp.float32),
        ],
        interpret=interpret,
    )(qkvT, qkvT, qkvT)


def _proj_ln_gate_body(y_ref, x_ref, wp_ref, bp_ref, g_ref, b_ref, wg_ref,
                       bg_ref, x1_ref, ln2_ref, lg_ref):
    att = jnp.dot(y_ref[:], wp_ref[:], preferred_element_type=jnp.float32)
    x1 = x_ref[:] + att + bp_ref[:]
    x1_ref[:] = x1
    m = jnp.mean(x1, axis=1, keepdims=True)
    xc = x1 - m
    v = jnp.mean(xc * xc, axis=1, keepdims=True)
    ln2 = xc * lax.rsqrt(v + 1e-5) * g_ref[:] + b_ref[:]
    ln2_ref[:] = ln2
    lg_ref[:] = (
        jnp.dot(ln2, wg_ref[:], preferred_element_type=jnp.float32) + bg_ref[:]
    )


def _proj_ln_gate(y, x2, Wproj, bproj, g2, b2, Wg, bg, interpret=False):
    return pl.pallas_call(
        _proj_ln_gate_body,
        grid=(T // RB,),
        in_specs=[
            pl.BlockSpec((RB, N), lambda i: (i, 0)),
            pl.BlockSpec((RB, N), lambda i: (i, 0)),
            pl.BlockSpec((N, N), lambda i: (0, 0)),
            pl.BlockSpec((1, N), lambda i: (0, 0)),
            pl.BlockSpec((1, N), lambda i: (0, 0)),
            pl.BlockSpec((1, N), lambda i: (0, 0)),
            pl.BlockSpec((N, E), lambda i: (0, 0)),
            pl.BlockSpec((1, E), lambda i: (0, 0)),
        ],
        out_specs=[
            pl.BlockSpec((RB, N), lambda i: (i, 0)),
            pl.BlockSpec((RB, N), lambda i: (i, 0)),
            pl.BlockSpec((RB, E), lambda i: (i, 0)),
        ],
        out_shape=[
            jax.ShapeDtypeStruct((T, N), jnp.float32),
            jax.ShapeDtypeStruct((T, N), jnp.float32),
            jax.ShapeDtypeStruct((T, E), jnp.float32),
        ],
        interpret=interpret,
    )(y, x2, Wproj, bproj.reshape(1, N), g2.reshape(1, N), b2.reshape(1, N),
      Wg, bg.reshape(1, E))


def _top2(le):
    v1 = jnp.full((16,), -1e30, jnp.float32)
    i1 = jnp.zeros((16,), jnp.int32)
    for e in range(E):
        upd = le[e] > v1
        v1 = jnp.where(upd, le[e], v1)
        i1 = jnp.where(upd, e, i1)
    v2 = jnp.full((16,), -1e30, jnp.float32)
    i2 = jnp.zeros((16,), jnp.int32)
    for e in range(E):
        upd = (le[e] > v2) & (i1 != e)
        v2 = jnp.where(upd, le[e], v2)
        i2 = jnp.where(upd, e, i2)
    return v1, i1, v2, i2


def _gather_logits(lg_v, c):
    return [lg_v[pl.ds(c * 16 + e * T, 16)] for e in range(E)]


def _count_chunk(c, cnts, lg_v):
    le = _gather_logits(lg_v, c)
    _, i1, _, i2 = _top2(le)
    out = []
    for e in range(E):
        mask_e = (i1 == e) | (i2 == e)
        out.append(cnts[e] + plsc.all_reduce_population_count(mask_e))
    return tuple(out)


def _route_body(lg_hbm, ln2_hbm, xg_hbm, pg_hbm, cnt_hbm,
                lg_v, rows_v, pg_v, dA_v, dB_v, stage_v):
    nc = 2
    wid = lax.axis_index("s") * nc + lax.axis_index("c")
    my_c0 = wid * 4
    t0 = wid * 64

    pltpu.sync_copy(lg_hbm, lg_v)

    zero = jnp.zeros((16,), jnp.int32)
    init = (zero,) * E
    base = lax.fori_loop(0, my_c0, lambda c, a: _count_chunk(c, a, lg_v),
                         init)
    tot = lax.fori_loop(my_c0, T // 16,
                        lambda c, a: _count_chunk(c, a, lg_v), base)

    pb = []
    run = jnp.zeros((16,), jnp.int32)
    for e in range(E):
        pb.append(run)
        padded = lax.shift_left(
            lax.shift_right_logical(tot[e] + (MB - 1), 7), 7)
        run = run + padded

    cnts = list(base)
    lane = lax.iota(jnp.int32, 16)
    for c in range(4):
        le = _gather_logits(lg_v, my_c0 + c)
        v1, i1, v2, i2 = _top2(le)
        m = le[0]
        for e in range(1, E):
            m = jnp.maximum(m, le[e])
        ssum = jnp.zeros((16,), jnp.float32)
        for e in range(E):
            ssum = ssum + jnp.exp(le[e] - m)
        p1 = jnp.exp(v1 - m) / ssum
        p2 = jnp.exp(v2 - m) / ssum
        rank1 = jnp.zeros((16,), jnp.int32)
        rank2 = jnp.zeros((16,), jnp.int32)
        for e in range(E):
            me1 = i1 == e
            me2 = i2 == e
            mask_e = me1 | me2
            pos = cnts[e] + plsc.cumsum(mask_e.astype(jnp.int32)) - 1
            rank1 = jnp.where(me1, pos, rank1)
            rank2 = jnp.where(me2, pos, rank2)
            cnts[e] = cnts[e] + plsc.all_reduce_population_count(mask_e)
        d1 = rank1
        d2 = rank2
        for e in range(E):
            d1 = d1 + jnp.where(i1 == e, pb[e], zero)
            d2 = d2 + jnp.where(i2 == e, pb[e], zero)
        sl = pl.ds(c * 16, 16)
        dA_v[sl] = d1
        dB_v[sl] = d2
        row = c * 16 + lane
        col = jnp.zeros((16,), jnp.int32)
        plsc.store_scatter(pg_v, [row, col], p1)
        plsc.store_scatter(pg_v, [row + 64, col], p2)

    pltpu.sync_copy(ln2_hbm.at[pl.ds(t0, 64)], rows_v)
    pltpu.sync_copy(rows_v, xg_hbm.at[dA_v])
    pltpu.sync_copy(rows_v, xg_hbm.at[dB_v])
    pltpu.sync_copy(pg_v.at[pl.ds(0, 64)], pg_hbm.at[dA_v])
    pltpu.sync_copy(pg_v.at[pl.ds(64, 64)], pg_hbm.at[dB_v])

    @pl.when(wid == 0)
    def _():
        cl = jnp.zeros((16,), jnp.int32)
        for e in range(E):
            cl = cl + jnp.where(lane == e, tot[e], zero)
        stage_v[:] = cl
        pltpu.sync_copy(stage_v, cnt_hbm)


def _route_sc(logits, ln2):
    mesh = plsc.VectorSubcoreMesh(core_axis_name="c", subcore_axis_name="s")
    f = pl.kernel(
        _route_body,
        out_type=[
            jax.ShapeDtypeStruct((XG_ROWS, N), jnp.float32),
            jax.ShapeDtypeStruct((XG_ROWS, 128), jnp.float32),
            jax.ShapeDtypeStruct((16,), jnp.int32),
        ],
        mesh=mesh,
        compiler_params=pltpu.CompilerParams(needs_layout_passes=False),
        scratch_types=[
            pltpu.VMEM((T * E,), jnp.float32),
            pltpu.VMEM((64, N), jnp.float32),
            pltpu.VMEM((128, 128), jnp.float32),
            pltpu.VMEM((64,), jnp.int32),
            pltpu.VMEM((64,), jnp.int32),
            pltpu.VMEM((16,), jnp.int32),
        ],
    )
    return f(logits.T.reshape(E * T), ln2)


def _moe_body(be_ref, bv_ref, bo_ref, xg_ref, pg_ref, wfc_ref, bfc_ref,
              wfp_ref, bfp_ref, x1_ref, o_ref):
    b = pl.program_id(0)

    @pl.when(b == 0)
    def _():
        o_ref[:] = x1_ref[:]

    valid = bv_ref[b]
    rows = lax.broadcasted_iota(jnp.int32, (MB, 1), 0)
    mask = rows < valid
    xb = jnp.where(mask, xg_ref[:], 0.0)
    prob = jnp.where(mask, pg_ref[0][:, 0:1], 0.0)
    h = jnp.dot(xb, wfc_ref[0], preferred_element_type=jnp.float32) + bfc_ref[0]
    h = jax.nn.gelu(h, approximate=True)
    o = jnp.dot(h, wfp_ref[0], preferred_element_type=jnp.float32) + bfp_ref[0]
    o = o * prob
    start = pl.multiple_of(bo_ref[b], MB)
    sl = pl.ds(start, MB)
    o_ref[sl, :] = o_ref[sl, :] + o


def _moe_ffn(be, bv, bo, xg, pg, Wfc, bfc, Wfp, bfp, x1, interpret=False):
    grid_spec = pltpu.PrefetchScalarGridSpec(
        num_scalar_prefetch=3,
        grid=(NBLK,),
        in_specs=[
            pl.BlockSpec((MB, N), lambda b, be, bv, bo: (b, 0)),
            pl.BlockSpec((1, MB, 128), lambda b, be, bv, bo: (b, 0, 0)),
            pl.BlockSpec((1, N, FF), lambda b, be, bv, bo: (be[b], 0, 0)),
            pl.BlockSpec((1, 1, FF), lambda b, be, bv, bo: (be[b], 0, 0)),
            pl.BlockSpec((1, FF, N), lambda b, be, bv, bo: (be[b], 0, 0)),
            pl.BlockSpec((1, 1, N), lambda b, be, bv, bo: (be[b], 0, 0)),
            pl.BlockSpec((T, N), lambda b, be, bv, bo: (0, 0)),
        ],
        out_specs=pl.BlockSpec((T, N), lambda b, be, bv, bo: (0, 0)),
    )
    return pl.pallas_call(
        _moe_body,
        grid_spec=grid_spec,
        out_shape=jax.ShapeDtypeStruct((T, N), jnp.float32),
        compiler_params=pltpu.CompilerParams(vmem_limit_bytes=100663296),
        interpret=interpret,
    )(be, bv, bo, xg, pg.reshape(NBLK, MB, 128), Wfc, bfc.reshape(E, 1, FF),
      Wfp, bfp.reshape(E, 1, N), x1)


def _block_tables(cnt16):
    cnt = cnt16[:E]
    nblk = (cnt + (MB - 1)) // MB
    ends = jnp.cumsum(nblk)
    starts = ends - nblk
    b_idx = jnp.arange(NBLK, dtype=jnp.int32)
    be = jnp.sum((b_idx[:, None] >= ends[None, :]).astype(jnp.int32), axis=1)
    be = jnp.minimum(be, E - 1).astype(jnp.int32)
    bo_blocks = b_idx - starts[be]
    bv = jnp.clip(cnt[be] - bo_blocks * MB, 0, MB).astype(jnp.int32)
    bo = jnp.clip(bo_blocks * MB, 0, T - MB).astype(jnp.int32)
    return be, bv, bo


@jax.jit
def kernel(x, g1, b1, Wqkv, bqkv, Wproj, bproj, g2, b2, Wg, bg, Wfc, bfc,
           Wfp, bfp):
    x2 = x.reshape(T, N)
    qkv = _ln_qkv(x2, g1, b1, Wqkv, bqkv)
    qkvT = qkv.reshape(T, 3 * H, DH).transpose(1, 0, 2)
    y4 = _attn(qkvT)
    y = y4.transpose(1, 0, 2).reshape(T, N)
    x1, ln2, logits = _proj_ln_gate(y, x2, Wproj, bproj, g2, b2, Wg, bg)
    xg, pg, cnt16 = _route_sc(logits, ln2)
    be, bv, bo = _block_tables(cnt16)
    out = _moe_ffn(be, bv, bo, xg, pg, Wfc, bfc, Wfp, bfp, x1)
    return out.reshape(x.shape)
